# baseline (speedup 1.0000x reference)
import functools

import jax
import jax.numpy as jnp
from jax import lax
from jax.experimental import pallas as pl
from jax.experimental.pallas import tpu as pltpu
from jax.experimental.pallas import tpu_sc as plsc

NUM_SYMBOLS = 96
SYM_DIM = 256
B, L = 4096, 50

_info = plsc.get_sparse_core_info()
NC, NS = _info.num_cores, _info.num_subcores
NW = NC * NS
ROWS_W = B // NW
NBUF = 4
LOOK = 2
NGROUP = ROWS_W // NBUF

_mesh = plsc.VectorSubcoreMesh(core_axis_name="c", subcore_axis_name="s")


@functools.partial(
    pl.kernel,
    mesh=_mesh,
    out_type=(
        jax.ShapeDtypeStruct((NW * NUM_SYMBOLS, SYM_DIM), jnp.float32),
        jax.ShapeDtypeStruct((B, L), jnp.int32),
    ),
    scratch_types=[
        pltpu.VMEM((NUM_SYMBOLS, SYM_DIM), jnp.float32),
        pltpu.VMEM((ROWS_W, L), jnp.int32),
    ],
)
def _prep_sc(table_hbm, idx_hbm, rep_out, idx_out, table_v, idx_v):
    wid = lax.axis_index("s") * NC + lax.axis_index("c")
    base = wid * ROWS_W
    pltpu.sync_copy(table_hbm,
                    rep_out.at[pl.ds(wid * NUM_SYMBOLS, NUM_SYMBOLS)])
    pltpu.sync_copy(idx_hbm.at[pl.ds(base, ROWS_W)], idx_v)
    off = jnp.full((16,), 0, jnp.int32) + wid * NUM_SYMBOLS
    cols = (0, 16, 32, L - 16)

    def offset_row(r, carry):
        vals = [idx_v[r, pl.ds(c, 16)] + off for c in cols]
        for c, v in zip(cols, vals):
            idx_v[r, pl.ds(c, 16)] = v
        return carry

    lax.fori_loop(0, ROWS_W, offset_row, 0)
    pltpu.sync_copy(idx_v, idx_out.at[pl.ds(base, ROWS_W)])


@functools.partial(
    pl.kernel,
    mesh=_mesh,
    out_type=jax.ShapeDtypeStruct((B, L, SYM_DIM), jnp.float32),
    scratch_types=[
        pltpu.VMEM((ROWS_W, L), jnp.int32),
        pltpu.VMEM((NBUF, L, SYM_DIM), jnp.float32),
        pltpu.SemaphoreType.DMA((NBUF,)),
        pltpu.SemaphoreType.DMA((NBUF,)),
    ],
)
def _gather_sc(rep_hbm, idx_hbm, out_hbm, idx_v, rows_v, gsem, ssem):
    wid = lax.axis_index("s") * NC + lax.axis_index("c")
    base = wid * ROWS_W
    pltpu.sync_copy(idx_hbm.at[pl.ds(base, ROWS_W)], idx_v)

    def gather(r, b):
        return pltpu.make_async_copy(
            rep_hbm.at[idx_v.at[r]], rows_v.at[b], gsem.at[b])

    def scatter(r, b):
        return pltpu.make_async_copy(
            rows_v.at[b], out_hbm.at[base + r], ssem.at[b])

    for b in range(LOOK):
        gather(b, b).start()

    def body(g, carry):
        for b in range(NBUF):
            r = g * NBUF + b
            rn = r + LOOK
            bn = (b + LOOK) % NBUF
            @pl.when(jnp.logical_and(rn >= NBUF, rn < ROWS_W))
            def _():
                scatter(rn - NBUF, bn).wait()
            @pl.when(rn < ROWS_W)
            def _():
                gather(rn, bn).start()
            gather(r, b).wait()
            scatter(r, b).start()
        return carry

    lax.fori_loop(0, NGROUP, body, 0)
    for b in range(NBUF):
        scatter(ROWS_W - NBUF + b, b).wait()


def kernel(QR, symbols):
    rep, QRo = _prep_sc(symbols, QR)
    return _gather_sc(rep, QRo)
